# Initial kernel scaffold; baseline (speedup 1.0000x reference)
import functools

import jax
import jax.numpy as jnp
from jax import lax
from jax.experimental import pallas as pl
from jax.experimental.pallas import tpu as pltpu
from jax.experimental.pallas import tpu_sc as plsc

L = 16
NC = 2
NS = 16
NW = NC * NS


def _build_mask_chunks(b_v, diff_v, br, lo_row, hi_row, scale, n_chunks):
    zero = jnp.zeros((L,), jnp.int32)
    for c in range(n_chunks):
        diff_v[pl.ds(c * L, L)] = zero
    for h in range(2):
        sl = pl.ds(h * L, L)
        lo = b_v[br, lo_row, sl]
        hi = b_v[br, hi_row, sl]
        lab = b_v[br, 4, sl]
        cnd = jnp.where(lab != 0.0, jnp.int32(1), jnp.int32(0))
        loi = (lo * scale).astype(jnp.int32)
        hii = (hi * scale).astype(jnp.int32)
        plsc.addupdate_scatter(diff_v, [loi], cnd)
        plsc.addupdate_scatter(diff_v, [hii], -cnd)
    chunks = []
    carry = zero
    total = zero
    for c in range(n_chunks):
        dv = diff_v[pl.ds(c * L, L)]
        cs = plsc.cumsum(dv) + carry
        carry = carry + jnp.broadcast_to(jnp.sum(dv), (L,))
        mi = cs > 0
        chunks.append(mi)
        total = total + mi.astype(jnp.int32)
    return chunks, total


def _make_sc_call(H, W, scale):
    rows_per = H // NW
    rch = H // L
    cch = W // L
    mesh = plsc.VectorSubcoreMesh(core_axis_name="c", subcore_axis_name="s")

    @functools.partial(
        pl.kernel,
        out_type=(
            jax.ShapeDtypeStruct((H, W), jnp.float32),
            jax.ShapeDtypeStruct((H, W), jnp.float32),
            jax.ShapeDtypeStruct((2, L), jnp.float32),
        ),
        mesh=mesh,
        scratch_types=[
            pltpu.VMEM((2, 5, 32), jnp.float32),
            pltpu.VMEM((max(rch, cch) * L,), jnp.int32),
            pltpu.VMEM((H,), jnp.int32),
            pltpu.VMEM((rows_per, W), jnp.float32),
            pltpu.VMEM((2, L), jnp.float32),
        ],
    )
    def sc_body(boxes_hbm, mask_pre_hbm, mask_cur_hbm, norms_hbm,
                b_v, diff_v, rowm_v, out_v, norms_v):
        wid = lax.axis_index("s") * NC + lax.axis_index("c")
        base = wid * rows_per
        pltpu.sync_copy(boxes_hbm, b_v)
        for br in range(2):
            rmask, rtot = _build_mask_chunks(b_v, diff_v, br, 1, 3, scale, rch)
            for c in range(rch):
                rowm_v[pl.ds(c * L, L)] = rmask[c].astype(jnp.int32)
            cmask, ctot = _build_mask_chunks(b_v, diff_v, br, 0, 2, scale, cch)
            colf = [m.astype(jnp.float32) for m in cmask]
            prod = (jnp.broadcast_to(jnp.sum(rtot), (L,))
                    * jnp.broadcast_to(jnp.sum(ctot), (L,)) * 2)
            norms_v[br, pl.ds(0, L)] = jnp.where(
                prod > 0, prod.astype(jnp.float32), jnp.float32(1.0))
            zrow = jnp.zeros((L,), jnp.float32)
            for rr in range(rows_per):
                on = rowm_v[base + rr] > 0
                for c in range(cch):
                    out_v[rr, pl.ds(c * L, L)] = jnp.where(on, colf[c], zrow)
            dst = mask_pre_hbm if br == 0 else mask_cur_hbm
            pltpu.sync_copy(out_v, dst.at[pl.ds(base, rows_per)])

        @pl.when(wid == 0)
        def _():
            pltpu.sync_copy(norms_v, norms_hbm)

    return sc_body


def _pack_boxes(gt_boxes, num_boxes):
    bb = gt_boxes[0]
    gate = (num_boxes[0] > 0).astype(jnp.float32)
    lab = bb[:, 4:5] * gate
    bt = jnp.concatenate([bb[:, :4], lab], axis=1).T
    return jnp.pad(bt, ((0, 0), (0, 32 - bt.shape[1])))


def kernel(im_data, feature, gt_boxes_pre, num_boxes_pre, gt_boxes_cur,
           num_boxes_cur):
    H, W = feature.shape[2], feature.shape[3]
    H_img = im_data.shape[2]
    scale = float(H) / float(H_img)
    boxes = jnp.stack([_pack_boxes(gt_boxes_pre, num_boxes_pre),
                       _pack_boxes(gt_boxes_cur, num_boxes_cur)])
    mask_pre, mask_cur, norms = _make_sc_call(H, W, scale)(boxes)
    return (mask_pre[None, None], norms[0, 0],
            mask_cur[None, None], norms[1, 0])


# speedup vs baseline: 5.9904x; 5.9904x over previous
import functools

import jax
import jax.numpy as jnp
from jax import lax
from jax.experimental import pallas as pl
from jax.experimental.pallas import tpu as pltpu
from jax.experimental.pallas import tpu_sc as plsc

L = 16
NC = 2
NS = 16
NW = NC * NS


def _build_mask_chunks(b_v, diff_v, br, lo_row, hi_row, scale, n_chunks):
    zero = jnp.zeros((L,), jnp.int32)
    for c in range(n_chunks):
        diff_v[pl.ds(c * L, L)] = zero
    for h in range(2):
        off = (br * 5) * 32 + h * L
        lo = b_v[pl.ds(off + lo_row * 32, L)]
        hi = b_v[pl.ds(off + hi_row * 32, L)]
        lab = b_v[pl.ds(off + 4 * 32, L)]
        cnd = jnp.where(lab != 0.0, jnp.int32(1), jnp.int32(0))
        loi = (lo * scale).astype(jnp.int32)
        hii = (hi * scale).astype(jnp.int32)
        plsc.addupdate_scatter(diff_v, [loi], cnd)
        plsc.addupdate_scatter(diff_v, [hii], -cnd)
    chunks = []
    carry = zero
    total = zero
    for c in range(n_chunks):
        dv = diff_v[pl.ds(c * L, L)]
        cs = plsc.cumsum(dv) + carry
        carry = carry + jnp.broadcast_to(jnp.sum(dv), (L,))
        mi = cs > 0
        chunks.append(mi)
        total = total + mi.astype(jnp.int32)
    return chunks, total


def _make_sc_call(H, W, scale):
    rows_per = H // NW
    rch = H // L
    cch = W // L
    mesh = plsc.VectorSubcoreMesh(core_axis_name="c", subcore_axis_name="s")

    @functools.partial(
        pl.kernel,
        out_type=(
            jax.ShapeDtypeStruct((H, W), jnp.float32),
            jax.ShapeDtypeStruct((H, W), jnp.float32),
            jax.ShapeDtypeStruct((2 * L,), jnp.float32),
        ),
        mesh=mesh,
        compiler_params=pltpu.CompilerParams(needs_layout_passes=False),
        scratch_types=[
            pltpu.VMEM((2 * 5 * 32,), jnp.float32),
            pltpu.VMEM((max(rch, cch) * L,), jnp.int32),
            pltpu.VMEM((H + L,), jnp.int32),
            pltpu.VMEM((rows_per, W), jnp.float32),
            pltpu.VMEM((2 * L,), jnp.float32),
        ],
    )
    def sc_body(boxes_hbm, mask_pre_hbm, mask_cur_hbm, norms_hbm,
                b_v, diff_v, rowm_v, out_v, norms_v):
        wid = lax.axis_index("s") * NC + lax.axis_index("c")
        base = wid * rows_per
        pltpu.sync_copy(boxes_hbm, b_v)
        for br in range(2):
            rmask, rtot = _build_mask_chunks(b_v, diff_v, br, 1, 3, scale, rch)
            for c in range(rch):
                rowm_v[pl.ds(c * L, L)] = rmask[c].astype(jnp.int32)
            cmask, ctot = _build_mask_chunks(b_v, diff_v, br, 0, 2, scale, cch)
            colf = [m.astype(jnp.float32) for m in cmask]
            prod = (jnp.broadcast_to(jnp.sum(rtot), (L,))
                    * jnp.broadcast_to(jnp.sum(ctot), (L,)) * 2)
            norms_v[pl.ds(br * L, L)] = jnp.where(
                prod > 0, prod.astype(jnp.float32), jnp.float32(1.0))
            zrow = jnp.zeros((L,), jnp.float32)
            myrows = rowm_v[pl.ds(base, L)]
            for rr in range(rows_per):
                on = myrows[rr] > 0
                for c in range(cch):
                    out_v[rr, pl.ds(c * L, L)] = jnp.where(on, colf[c], zrow)
            dst = mask_pre_hbm if br == 0 else mask_cur_hbm
            pltpu.sync_copy(out_v, dst.at[pl.ds(base, rows_per)])

        @pl.when(wid == 0)
        def _():
            pltpu.sync_copy(norms_v, norms_hbm)

    return sc_body


def _pack_boxes(gt_boxes, num_boxes):
    bb = gt_boxes[0]
    gate = (num_boxes[0] > 0).astype(jnp.float32)
    lab = bb[:, 4:5] * gate
    bt = jnp.concatenate([bb[:, :4], lab], axis=1).T
    return jnp.pad(bt, ((0, 0), (0, 32 - bt.shape[1])))


def kernel(im_data, feature, gt_boxes_pre, num_boxes_pre, gt_boxes_cur,
           num_boxes_cur):
    H, W = feature.shape[2], feature.shape[3]
    H_img = im_data.shape[2]
    scale = float(H) / float(H_img)
    boxes = jnp.stack([_pack_boxes(gt_boxes_pre, num_boxes_pre),
                       _pack_boxes(gt_boxes_cur, num_boxes_cur)]).reshape(-1)
    mask_pre, mask_cur, norms = _make_sc_call(H, W, scale)(boxes)
    return (mask_pre[None, None], norms[0],
            mask_cur[None, None], norms[L])


# speedup vs baseline: 6.1231x; 1.0222x over previous
import functools

import jax
import jax.numpy as jnp
from jax import lax
from jax.experimental import pallas as pl
from jax.experimental.pallas import tpu as pltpu
from jax.experimental.pallas import tpu_sc as plsc

L = 16
NC = 2
NS = 16
N_BOX = 20
B_STRIDE = 128


def _make_sc_call(H, W, scale):
    rows_per = H // NS
    rch = H // L
    cch = W // L
    mesh = plsc.VectorSubcoreMesh(core_axis_name="c", subcore_axis_name="s")

    @functools.partial(
        pl.kernel,
        out_type=(
            jax.ShapeDtypeStruct((H, W), jnp.float32),
            jax.ShapeDtypeStruct((H, W), jnp.float32),
            jax.ShapeDtypeStruct((2 * L,), jnp.float32),
        ),
        mesh=mesh,
        compiler_params=pltpu.CompilerParams(needs_layout_passes=False),
        scratch_types=[
            pltpu.VMEM((2 * B_STRIDE,), jnp.float32),
            pltpu.VMEM((L,), jnp.int32),
            pltpu.VMEM((max(rch, cch) * L,), jnp.int32),
            pltpu.VMEM((H + L,), jnp.int32),
            pltpu.VMEM((rows_per, W), jnp.float32),
            pltpu.VMEM((L,), jnp.float32),
            pltpu.SemaphoreType.DMA,
        ],
    )
    def sc_body(bp_hbm, bc_hbm, np_hbm, nc_hbm,
                mask_pre_hbm, mask_cur_hbm, norms_hbm,
                b_v, nb_v, diff_v, rowm_v, out_v, norms_v, sem):
        br = lax.axis_index("c")
        sid = lax.axis_index("s")
        rbase = sid * rows_per
        cp0 = pltpu.async_copy(bp_hbm, b_v.at[pl.ds(0, 5 * N_BOX)], sem)
        cp1 = pltpu.async_copy(bc_hbm, b_v.at[pl.ds(B_STRIDE, 5 * N_BOX)], sem)
        cp2 = pltpu.async_copy(np_hbm, nb_v.at[pl.ds(0, 1)], sem)
        cp3 = pltpu.async_copy(nc_hbm, nb_v.at[pl.ds(8, 1)], sem)
        cp0.wait()
        cp1.wait()
        cp2.wait()
        cp3.wait()

        nbv = nb_v[pl.ds(0, L)]
        gate = jnp.where(br == 0, nbv[0] > 0, nbv[8] > 0)
        b_off = br * B_STRIDE
        iota = lax.iota(jnp.int32, L)
        i5 = iota * 5

        def build(lo_f, hi_f, n_chunks):
            zero = jnp.zeros((L,), jnp.int32)
            for c in range(n_chunks):
                diff_v[pl.ds(c * L, L)] = zero
            for h in range((N_BOX + L - 1) // L):
                base = b_off + h * 5 * L
                lane_ok = iota < (N_BOX - h * L)
                idx_lim = 2 * B_STRIDE - 1
                lo_i = jnp.minimum(i5 + (base + lo_f), idx_lim)
                hi_i = jnp.minimum(i5 + (base + hi_f), idx_lim)
                lab_i = jnp.minimum(i5 + (base + 4), idx_lim)
                lo = plsc.load_gather(b_v, [lo_i])
                hi = plsc.load_gather(b_v, [hi_i])
                lab = plsc.load_gather(b_v, [lab_i])
                cnd = jnp.where((lab != 0.0) & lane_ok & gate,
                                jnp.int32(1), jnp.int32(0))
                loi = jnp.clip((lo * scale).astype(jnp.int32),
                               0, n_chunks * L - 1)
                hii = jnp.clip((hi * scale).astype(jnp.int32),
                               0, n_chunks * L - 1)
                plsc.addupdate_scatter(diff_v, [loi], cnd)
                plsc.addupdate_scatter(diff_v, [hii], -cnd)
            chunks = []
            carry = zero
            total = zero
            for c in range(n_chunks):
                dv = diff_v[pl.ds(c * L, L)]
                cs = plsc.cumsum(dv) + carry
                carry = carry + jnp.broadcast_to(jnp.sum(dv), (L,))
                mi = cs > 0
                chunks.append(mi)
                total = total + mi.astype(jnp.int32)
            return chunks, total

        rmask, rtot = build(1, 3, rch)
        for c in range(rch):
            rowm_v[pl.ds(c * L, L)] = rmask[c].astype(jnp.int32)
        cmask, ctot = build(0, 2, cch)
        colf = [m.astype(jnp.float32) for m in cmask]

        prod = (jnp.broadcast_to(jnp.sum(rtot), (L,))
                * jnp.broadcast_to(jnp.sum(ctot), (L,)) * 2)
        norms_v[pl.ds(0, L)] = jnp.where(
            prod > 0, prod.astype(jnp.float32), jnp.float32(1.0))

        zrow = jnp.zeros((L,), jnp.float32)
        myrows = rowm_v[pl.ds(rbase, L)]
        for rr in range(rows_per):
            on = myrows[rr] > 0
            for c in range(cch):
                out_v[rr, pl.ds(c * L, L)] = jnp.where(on, colf[c], zrow)

        @pl.when(br == 0)
        def _():
            pltpu.sync_copy(out_v, mask_pre_hbm.at[pl.ds(rbase, rows_per)])

            @pl.when(sid == 0)
            def _():
                pltpu.sync_copy(norms_v, norms_hbm.at[pl.ds(0, L)])

        @pl.when(br == 1)
        def _():
            pltpu.sync_copy(out_v, mask_cur_hbm.at[pl.ds(rbase, rows_per)])

            @pl.when(sid == 0)
            def _():
                pltpu.sync_copy(norms_v, norms_hbm.at[pl.ds(L, L)])

    return sc_body


def kernel(im_data, feature, gt_boxes_pre, num_boxes_pre, gt_boxes_cur,
           num_boxes_cur):
    H, W = feature.shape[2], feature.shape[3]
    H_img = im_data.shape[2]
    scale = float(H) / float(H_img)
    bp = gt_boxes_pre.reshape(-1)
    bc = gt_boxes_cur.reshape(-1)
    mask_pre, mask_cur, norms = _make_sc_call(H, W, scale)(
        bp, bc, num_boxes_pre, num_boxes_cur)
    return (mask_pre[None, None], norms[0],
            mask_cur[None, None], norms[L])


# speedup vs baseline: 6.1839x; 1.0099x over previous
import functools

import jax
import jax.numpy as jnp
from jax import lax
from jax.experimental import pallas as pl
from jax.experimental.pallas import tpu as pltpu
from jax.experimental.pallas import tpu_sc as plsc

L = 16
NC = 2
NS = 16
N_BOX = 20
B_STRIDE = 128


def _make_sc_call(H, W, scale):
    rows_per = H // NS
    rch = H // L
    cch = W // L
    mesh = plsc.VectorSubcoreMesh(core_axis_name="c", subcore_axis_name="s")

    @functools.partial(
        pl.kernel,
        out_type=(
            jax.ShapeDtypeStruct((1, 1, H, W), jnp.float32),
            jax.ShapeDtypeStruct((1, 1, H, W), jnp.float32),
            jax.ShapeDtypeStruct((2 * L,), jnp.float32),
        ),
        mesh=mesh,
        compiler_params=pltpu.CompilerParams(needs_layout_passes=False),
        scratch_types=[
            pltpu.VMEM((2 * B_STRIDE,), jnp.float32),
            pltpu.VMEM((L,), jnp.int32),
            pltpu.VMEM((max(rch, cch) * L,), jnp.int32),
            pltpu.VMEM((H + L,), jnp.int32),
            pltpu.VMEM((rows_per, W), jnp.float32),
            pltpu.VMEM((L,), jnp.float32),
            pltpu.SemaphoreType.DMA,
        ],
    )
    def sc_body(bp_hbm, bc_hbm, np_hbm, nc_hbm,
                mask_pre_hbm, mask_cur_hbm, norms_hbm,
                b_v, nb_v, diff_v, rowm_v, out_v, norms_v, sem):
        br = lax.axis_index("c")
        sid = lax.axis_index("s")
        rbase = sid * rows_per
        cp0 = pltpu.async_copy(bp_hbm, b_v.at[pl.ds(0, 5 * N_BOX)], sem)
        cp1 = pltpu.async_copy(bc_hbm, b_v.at[pl.ds(B_STRIDE, 5 * N_BOX)], sem)
        cp2 = pltpu.async_copy(np_hbm, nb_v.at[pl.ds(0, 1)], sem)
        cp3 = pltpu.async_copy(nc_hbm, nb_v.at[pl.ds(8, 1)], sem)
        cp0.wait()
        cp1.wait()
        cp2.wait()
        cp3.wait()

        nbv = nb_v[pl.ds(0, L)]
        gate = jnp.where(br == 0, nbv[0] > 0, nbv[8] > 0)
        b_off = br * B_STRIDE
        iota = lax.iota(jnp.int32, L)
        i5 = iota * 5

        def build(lo_f, hi_f, n_chunks):
            zero = jnp.zeros((L,), jnp.int32)
            for c in range(n_chunks):
                diff_v[pl.ds(c * L, L)] = zero
            for h in range((N_BOX + L - 1) // L):
                base = b_off + h * 5 * L
                lane_ok = iota < (N_BOX - h * L)
                idx_lim = 2 * B_STRIDE - 1
                lo_i = jnp.minimum(i5 + (base + lo_f), idx_lim)
                hi_i = jnp.minimum(i5 + (base + hi_f), idx_lim)
                lab_i = jnp.minimum(i5 + (base + 4), idx_lim)
                lo = plsc.load_gather(b_v, [lo_i])
                hi = plsc.load_gather(b_v, [hi_i])
                lab = plsc.load_gather(b_v, [lab_i])
                cnd = jnp.where((lab != 0.0) & lane_ok & gate,
                                jnp.int32(1), jnp.int32(0))
                loi = jnp.clip((lo * scale).astype(jnp.int32),
                               0, n_chunks * L - 1)
                hii = jnp.clip((hi * scale).astype(jnp.int32),
                               0, n_chunks * L - 1)
                plsc.addupdate_scatter(diff_v, [loi], cnd)
                plsc.addupdate_scatter(diff_v, [hii], -cnd)
            chunks = []
            carry = zero
            total = zero
            for c in range(n_chunks):
                dv = diff_v[pl.ds(c * L, L)]
                cs = plsc.cumsum(dv) + carry
                carry = carry + jnp.broadcast_to(jnp.sum(dv), (L,))
                mi = cs > 0
                chunks.append(mi)
                total = total + mi.astype(jnp.int32)
            return chunks, total

        rmask, rtot = build(1, 3, rch)
        for c in range(rch):
            rowm_v[pl.ds(c * L, L)] = rmask[c].astype(jnp.int32)
        cmask, ctot = build(0, 2, cch)
        colf = [m.astype(jnp.float32) for m in cmask]

        prod = (jnp.broadcast_to(jnp.sum(rtot), (L,))
                * jnp.broadcast_to(jnp.sum(ctot), (L,)) * 2)
        norms_v[pl.ds(0, L)] = jnp.where(
            prod > 0, prod.astype(jnp.float32), jnp.float32(1.0))

        zrow = jnp.zeros((L,), jnp.float32)
        myrows = rowm_v[pl.ds(rbase, L)]
        for rr in range(rows_per):
            on = myrows[rr] > 0
            for c in range(cch):
                out_v[rr, pl.ds(c * L, L)] = jnp.where(on, colf[c], zrow)

        @pl.when(br == 0)
        def _():
            pltpu.sync_copy(out_v,
                            mask_pre_hbm.at[0, 0, pl.ds(rbase, rows_per)])

            @pl.when(sid == 0)
            def _():
                pltpu.sync_copy(norms_v, norms_hbm.at[pl.ds(0, L)])

        @pl.when(br == 1)
        def _():
            pltpu.sync_copy(out_v,
                            mask_cur_hbm.at[0, 0, pl.ds(rbase, rows_per)])

            @pl.when(sid == 0)
            def _():
                pltpu.sync_copy(norms_v, norms_hbm.at[pl.ds(L, L)])

    return sc_body


def kernel(im_data, feature, gt_boxes_pre, num_boxes_pre, gt_boxes_cur,
           num_boxes_cur):
    H, W = feature.shape[2], feature.shape[3]
    H_img = im_data.shape[2]
    scale = float(H) / float(H_img)
    bp = gt_boxes_pre.reshape(-1)
    bc = gt_boxes_cur.reshape(-1)
    mask_pre, mask_cur, norms = _make_sc_call(H, W, scale)(
        bp, bc, num_boxes_pre, num_boxes_cur)
    return (mask_pre, norms[0], mask_cur, norms[L])
